# Initial kernel scaffold; baseline (speedup 1.0000x reference)
import jax
import jax.numpy as jnp
from jax.experimental import pallas as pl


def kernel(x, adj_t, W1, b1, gammas, bn_betas, Wc, W2, b2):
    raise NotImplementedError("write your pallas kernel here")


# speedup vs baseline: 8.1667x; 8.1667x over previous
import functools

import jax
import jax.numpy as jnp
from jax import lax
from jax.experimental import pallas as pl
from jax.experimental.pallas import tpu as pltpu
from jax.experimental.pallas import tpu_sc as plsc

N_NODES = 10000
F = 128
E_EDGES = 320000
L_LAYERS = 4
ALPHA = 0.1

NC = 2
NS = 16
NW = NC * NS
CH = 128
CPT = 79
EPT = CPT * CH
EP = NW * EPT
ROWS_PAD = 10240
RPT = ROWS_PAD // NS

BLK = 400
NBLK = N_NODES // BLK

_mesh = plsc.VectorSubcoreMesh(core_axis_name="c", subcore_axis_name="s")


def _zero_fill(ref, nvec):
    z = jnp.zeros((16,), jnp.float32)

    def body(i, _):
        ref[i // 8, pl.ds((i % 8) * 16, 16)] = z
        return 0

    lax.fori_loop(0, nvec, body, 0)


@functools.partial(
    pl.kernel,
    out_type=jax.ShapeDtypeStruct((NC, ROWS_PAD), jnp.float32),
    mesh=_mesh,
    scratch_types=[
        pltpu.VMEM_SHARED((ROWS_PAD,), jnp.float32),
        pltpu.VMEM((CH,), jnp.float32),
        pltpu.VMEM((RPT,), jnp.float32),
        pltpu.VMEM((CH,), jnp.int32),
    ],
)
def _sc_degree(dst_hbm, out_hbm, acc, ones_v, zeros_v, dst_v):
    cid = lax.axis_index("c")
    sid = lax.axis_index("s")

    one = jnp.ones((16,), jnp.float32)
    zero = jnp.zeros((16,), jnp.float32)

    def fill(i, _):
        ones_v[pl.ds(i * 16, 16)] = one
        return 0

    lax.fori_loop(0, CH // 16, fill, 0)

    def zfill(i, _):
        zeros_v[pl.ds(i * 16, 16)] = zero
        return 0

    lax.fori_loop(0, RPT // 16, zfill, 0)

    rbase = sid * RPT
    pltpu.sync_copy(zeros_v, acc.at[pl.ds(rbase, RPT)])
    plsc.subcore_barrier()

    ebase = (cid * NS + sid) * EPT

    def body(i, _):
        off = ebase + i * CH
        pltpu.sync_copy(dst_hbm.at[pl.ds(off, CH)], dst_v)
        pltpu.sync_copy(ones_v, acc.at[dst_v], add=True)
        return 0

    lax.fori_loop(0, CPT, body, 0)
    plsc.subcore_barrier()
    pltpu.sync_copy(acc.at[pl.ds(rbase, RPT)], out_hbm.at[cid, pl.ds(rbase, RPT)])


@functools.partial(
    pl.kernel,
    out_type=jax.ShapeDtypeStruct((NC, ROWS_PAD, F), jnp.float32),
    mesh=_mesh,
    scratch_types=[
        pltpu.VMEM_SHARED((ROWS_PAD, F), jnp.float32),
        pltpu.VMEM((CH, F), jnp.float32),
        pltpu.VMEM((CH,), jnp.int32),
        pltpu.VMEM((CH,), jnp.int32),
        pltpu.VMEM((CH, F), jnp.float32),
        pltpu.SemaphoreType.DMA,
    ],
)
def _sc_propagate(hns_hbm, src_hbm, dst_hbm, out_hbm, acc, zbuf, src_v, dst_v,
                  rows, gsem):
    cid = lax.axis_index("c")
    sid = lax.axis_index("s")

    _zero_fill(zbuf, CH * 8)
    rbase = sid * RPT
    for r in range(0, RPT, CH):
        pltpu.sync_copy(zbuf, acc.at[pl.ds(rbase + r, CH)])
    plsc.subcore_barrier()

    ebase = (cid * NS + sid) * EPT

    def body(i, _):
        off = ebase + i * CH
        pltpu.sync_copy(src_hbm.at[pl.ds(off, CH)], src_v)
        pltpu.sync_copy(dst_hbm.at[pl.ds(off, CH)], dst_v)
        pltpu.async_copy(hns_hbm.at[src_v], rows, gsem).wait()
        pltpu.sync_copy(rows, acc.at[dst_v], add=True)
        return 0

    lax.fori_loop(0, CPT, body, 0)
    plsc.subcore_barrier()
    pltpu.sync_copy(acc.at[pl.ds(rbase, RPT)],
                    out_hbm.at[cid, pl.ds(rbase, RPT)])


def _dinv_body(d0_ref, d1_ref, o_ref):
    deg = d0_ref[...] + d1_ref[...] + 1.0
    o_ref[...] = lax.rsqrt(deg)


def _prelude_body(x_ref, w_ref, b_ref, h_ref, s_ref, q_ref):
    h = jnp.dot(x_ref[...], w_ref[...], preferred_element_type=jnp.float32)
    h = jnp.maximum(h + b_ref[...], 0.0)
    h_ref[...] = h

    @pl.when(pl.program_id(0) == 0)
    def _():
        s_ref[...] = jnp.zeros_like(s_ref)
        q_ref[...] = jnp.zeros_like(q_ref)

    s_ref[...] += jnp.sum(h, axis=0, keepdims=True)
    q_ref[...] += jnp.sum(h * h, axis=0, keepdims=True)


def _norm_body(h_ref, s_ref, q_ref, g_ref, b_ref, dinv_ref, hn_ref, hns_ref):
    inv_n = 1.0 / N_NODES
    mean = s_ref[...] * inv_n
    var = q_ref[...] * inv_n - mean * mean
    scale = lax.rsqrt(var + 1e-5) * g_ref[...]
    hn = (h_ref[...] - mean) * scale + b_ref[...]
    hn_ref[...] = hn
    hns_ref[...] = hn * dinv_ref[...]


def _combine_body(s0_ref, s1_ref, hn_ref, hns_ref, h0_ref, dinv_ref, wc_ref,
                  h_ref, s_ref, q_ref):
    ax = (s0_ref[...] + s1_ref[...] + hns_ref[...]) * dinv_ref[...]
    t = (1.0 - ALPHA) * ax + ALPHA * h0_ref[...]
    u = jnp.dot(t, wc_ref[...], preferred_element_type=jnp.float32)
    h = jnp.maximum(u, 0.0) + hn_ref[...]
    h_ref[...] = h

    @pl.when(pl.program_id(0) == 0)
    def _():
        s_ref[...] = jnp.zeros_like(s_ref)
        q_ref[...] = jnp.zeros_like(q_ref)

    s_ref[...] += jnp.sum(h, axis=0, keepdims=True)
    q_ref[...] += jnp.sum(h * h, axis=0, keepdims=True)


def _final_body(h_ref, w_ref, b_ref, o_ref):
    logits = jnp.dot(h_ref[...], w_ref[...],
                     preferred_element_type=jnp.float32) + b_ref[...]
    m = jnp.max(logits, axis=-1, keepdims=True)
    s = logits - m
    lse = jnp.log(jnp.sum(jnp.exp(s), axis=-1, keepdims=True))
    o_ref[...] = s - lse


def _row_spec(cols):
    return pl.BlockSpec((BLK, cols), lambda i: (i, 0))


def _bcast_spec(rows, cols):
    return pl.BlockSpec((rows, cols), lambda i: (0, 0))


_STATS_OUT = [
    jax.ShapeDtypeStruct((1, F), jnp.float32),
    jax.ShapeDtypeStruct((1, F), jnp.float32),
]

_dinv = pl.pallas_call(
    _dinv_body,
    grid=(),
    out_shape=jax.ShapeDtypeStruct((ROWS_PAD // F, F), jnp.float32),
)

_prelude = pl.pallas_call(
    _prelude_body,
    grid=(NBLK,),
    in_specs=[_row_spec(F), _bcast_spec(F, F), _bcast_spec(1, F)],
    out_specs=[_row_spec(F), _bcast_spec(1, F), _bcast_spec(1, F)],
    out_shape=[jax.ShapeDtypeStruct((N_NODES, F), jnp.float32)] + _STATS_OUT,
)

_norm = pl.pallas_call(
    _norm_body,
    grid=(NBLK,),
    in_specs=[_row_spec(F), _bcast_spec(1, F), _bcast_spec(1, F),
              _bcast_spec(1, F), _bcast_spec(1, F), _row_spec(1)],
    out_specs=[_row_spec(F), _row_spec(F)],
    out_shape=[jax.ShapeDtypeStruct((N_NODES, F), jnp.float32),
               jax.ShapeDtypeStruct((N_NODES, F), jnp.float32)],
)

_combine = pl.pallas_call(
    _combine_body,
    grid=(NBLK,),
    in_specs=[_row_spec(F), _row_spec(F), _row_spec(F), _row_spec(F),
              _row_spec(F), _row_spec(1), _bcast_spec(F, F)],
    out_specs=[_row_spec(F), _bcast_spec(1, F), _bcast_spec(1, F)],
    out_shape=[jax.ShapeDtypeStruct((N_NODES, F), jnp.float32)] + _STATS_OUT,
)

_final = pl.pallas_call(
    _final_body,
    grid=(NBLK,),
    in_specs=[_row_spec(F), _bcast_spec(F, 40), _bcast_spec(1, 40)],
    out_specs=_row_spec(40),
    out_shape=jax.ShapeDtypeStruct((N_NODES, 40), jnp.float32),
)


def kernel(x, adj_t, W1, b1, gammas, bn_betas, Wc, W2, b2):
    pad = EP - E_EDGES
    src_p = jnp.concatenate([adj_t[0], jnp.zeros((pad,), jnp.int32)])
    dst_p = jnp.concatenate([adj_t[1], jnp.full((pad,), N_NODES, jnp.int32)])

    deg2 = _sc_degree(dst_p)
    dinv2d = _dinv(deg2[0].reshape(ROWS_PAD // F, F),
                   deg2[1].reshape(ROWS_PAD // F, F))
    dinv_col = dinv2d.reshape(ROWS_PAD, 1)[:N_NODES]

    h, ssum, ssq = _prelude(x, W1, b1.reshape(1, F))
    h0 = h
    for l in range(L_LAYERS):
        hn, hns = _norm(h, ssum, ssq, gammas[l].reshape(1, F),
                        bn_betas[l].reshape(1, F), dinv_col)
        s2 = _sc_propagate(hns, src_p, dst_p)
        h, ssum, ssq = _combine(s2[0, :N_NODES], s2[1, :N_NODES], hn, hns,
                                h0, dinv_col, Wc[l])
    return _final(h, W2, b2.reshape(1, 40))
